# baseline (speedup 1.0000x reference)
import jax
import jax.numpy as jnp
from jax import lax
from jax.experimental import pallas as pl
from jax.experimental.pallas import tpu as pltpu
from jax.experimental.pallas import tpu_sc as plsc

_CUT = 5
_E = 320000
_D = 128

_NC = 2
_NS = 16
_NW = _NC * _NS
_LANES = 16

_PER_W = _E // _NW
_BLK = 400
_GRP = 80
_NG = _BLK // _GRP
_NB = _PER_W // _BLK


def _table_body(de_ref, w_ref, b_ref, t_ref):
    t_ref[...] = (
        jnp.dot(de_ref[...], w_ref[...], preferred_element_type=jnp.float32)
        + b_ref[...]
    )


def _gather_body(t_hbm, feat_hbm, out_hbm, feat_v, idx_v, rows_v, sem):
    c = lax.axis_index("c")
    s = lax.axis_index("s")
    wid = c * _NS + s

    def block(j, carry):
        base = wid * _PER_W + j * _BLK
        pltpu.sync_copy(feat_hbm.at[pl.ds(base, _BLK)], feat_v)
        vregs_per_grp = _GRP // _LANES
        for k in range(_BLK // _LANES):
            x = feat_v[pl.ds(k * _LANES, _LANES)]
            xi = jnp.clip(x, 1.0, _CUT - 1e-6).astype(jnp.int32)
            idx_v[k // vregs_per_grp, pl.ds((k % vregs_per_grp) * _LANES, _LANES)] = xi
        copies = []
        for g in range(_NG):
            copies.append(
                pltpu.async_copy(
                    t_hbm.at[idx_v.at[g]],
                    rows_v.at[pl.ds(g * _GRP, _GRP)],
                    sem,
                )
            )
        for cp in copies:
            cp.wait()
        pltpu.sync_copy(rows_v, out_hbm.at[pl.ds(base, _BLK)])
        return carry

    lax.fori_loop(0, _NB, block, 0)


def kernel(dist_feat, dist_embed, W_in, b_in):
    de_pad = jnp.zeros((8, _D), jnp.float32).at[: _CUT + 1].set(dist_embed)
    table = pl.pallas_call(
        _table_body,
        out_shape=jax.ShapeDtypeStruct((8, _D), jnp.float32),
    )(de_pad, W_in, b_in.reshape(1, _D))

    mesh = plsc.VectorSubcoreMesh(core_axis_name="c", subcore_axis_name="s")
    gather = pl.kernel(
        _gather_body,
        out_type=jax.ShapeDtypeStruct((_E, _D), jnp.float32),
        mesh=mesh,
        scratch_types=[
            pltpu.VMEM((_BLK,), jnp.float32),
            pltpu.VMEM((_NG, _GRP), jnp.int32),
            pltpu.VMEM((_BLK, _D), jnp.float32),
            pltpu.SemaphoreType.DMA,
        ],
    )
    return gather(table, dist_feat)


# speedup vs baseline: 9.1827x; 9.1827x over previous
import jax
import jax.numpy as jnp
from jax import lax
from jax.experimental import pallas as pl
from jax.experimental.pallas import tpu as pltpu
from jax.experimental.pallas import tpu_sc as plsc

_CUT = 5
_E = 320000
_D = 128

_NC = 2
_NS = 16
_NW = _NC * _NS
_LANES = 16

_PER_W = _E // _NW
_BLK = 400
_GRP = 80
_NG = _BLK // _GRP
_NB = _PER_W // _BLK


def _table_body(de_ref, w_ref, b_ref, t_ref):
    t = (
        jnp.dot(de_ref[...], w_ref[...], preferred_element_type=jnp.float32)
        + b_ref[...]
    )
    t_ref[...] = jnp.broadcast_to(t[None], (_NW, 8, _D))


def _gather_body(t_hbm, feat_hbm, out_hbm, feat_v, idx_v, rows_v, sem):
    c = lax.axis_index("c")
    s = lax.axis_index("s")
    wid = c * _NS + s

    def block(j, carry):
        base = wid * _PER_W + j * _BLK
        pltpu.sync_copy(feat_hbm.at[pl.ds(base, _BLK)], feat_v)
        vregs_per_grp = _GRP // _LANES
        for k in range(_BLK // _LANES):
            x = feat_v[pl.ds(k * _LANES, _LANES)]
            xi = jnp.clip(x, 1.0, _CUT - 1e-6).astype(jnp.int32) + wid * 8
            idx_v[k // vregs_per_grp, pl.ds((k % vregs_per_grp) * _LANES, _LANES)] = xi
        copies = []
        for g in range(_NG):
            copies.append(
                pltpu.async_copy(
                    t_hbm.at[idx_v.at[g]],
                    rows_v.at[pl.ds(g * _GRP, _GRP)],
                    sem,
                )
            )
        for cp in copies:
            cp.wait()
        pltpu.sync_copy(rows_v, out_hbm.at[pl.ds(base, _BLK)])
        return carry

    lax.fori_loop(0, _NB, block, 0)


def kernel(dist_feat, dist_embed, W_in, b_in):
    de_pad = jnp.zeros((8, _D), jnp.float32).at[: _CUT + 1].set(dist_embed)
    table = pl.pallas_call(
        _table_body,
        out_shape=jax.ShapeDtypeStruct((_NW, 8, _D), jnp.float32),
    )(de_pad, W_in, b_in.reshape(1, _D))
    table = table.reshape(_NW * 8, _D)

    mesh = plsc.VectorSubcoreMesh(core_axis_name="c", subcore_axis_name="s")
    gather = pl.kernel(
        _gather_body,
        out_type=jax.ShapeDtypeStruct((_E, _D), jnp.float32),
        mesh=mesh,
        scratch_types=[
            pltpu.VMEM((_BLK,), jnp.float32),
            pltpu.VMEM((_NG, _GRP), jnp.int32),
            pltpu.VMEM((_BLK, _D), jnp.float32),
            pltpu.SemaphoreType.DMA,
        ],
    )
    return gather(table, dist_feat)


# speedup vs baseline: 9.3543x; 1.0187x over previous
import jax
import jax.numpy as jnp
from jax import lax
from jax.experimental import pallas as pl
from jax.experimental.pallas import tpu as pltpu
from jax.experimental.pallas import tpu_sc as plsc

_CUT = 5
_E = 320000
_D = 128

_NC = 2
_NS = 16
_NW = _NC * _NS
_LANES = 16

_PER_W = _E // _NW
_BLK = 400
_GRP = 80
_NG = _BLK // _GRP
_NB = _PER_W // _BLK


def _table_body(de_ref, w_ref, b_ref, t_ref):
    t = (
        jnp.dot(de_ref[...], w_ref[...], preferred_element_type=jnp.float32)
        + b_ref[...]
    )
    t_ref[...] = jnp.broadcast_to(t[None], (_NW, 8, _D))


def _gather_body(t_hbm, feat_hbm, out_hbm, feat_v, idx_v, rows_v, sem_in, sem_out):
    c = lax.axis_index("c")
    s = lax.axis_index("s")
    wid = c * _NS + s
    vregs_per_grp = _GRP // _LANES

    def block(j, carry):
        b = j % 2
        base = wid * _PER_W + j * _BLK
        pltpu.sync_copy(feat_hbm.at[pl.ds(base, _BLK)], feat_v)
        for k in range(_BLK // _LANES):
            x = feat_v[pl.ds(k * _LANES, _LANES)]
            xi = jnp.clip(x, 1.0, _CUT - 1e-6).astype(jnp.int32) + wid * 8
            idx_v[k // vregs_per_grp, pl.ds((k % vregs_per_grp) * _LANES, _LANES)] = xi
        @pl.when(j >= 2)
        def _():
            prev = wid * _PER_W + (j - 2) * _BLK
            pltpu.make_async_copy(
                rows_v.at[b], out_hbm.at[pl.ds(prev, _BLK)], sem_out
            ).wait()

        copies = []
        for g in range(_NG):
            copies.append(
                pltpu.async_copy(
                    t_hbm.at[idx_v.at[g]],
                    rows_v.at[b, pl.ds(g * _GRP, _GRP)],
                    sem_in,
                )
            )
        for cp in copies:
            cp.wait()
        pltpu.async_copy(rows_v.at[b], out_hbm.at[pl.ds(base, _BLK)], sem_out)
        return carry

    lax.fori_loop(0, _NB, block, 0)
    for j in (_NB - 2, _NB - 1):
        b = j % 2
        base = wid * _PER_W + j * _BLK
        pltpu.make_async_copy(
            rows_v.at[b], out_hbm.at[pl.ds(base, _BLK)], sem_out
        ).wait()


def kernel(dist_feat, dist_embed, W_in, b_in):
    de_pad = jnp.zeros((8, _D), jnp.float32).at[: _CUT + 1].set(dist_embed)
    table = pl.pallas_call(
        _table_body,
        out_shape=jax.ShapeDtypeStruct((_NW, 8, _D), jnp.float32),
    )(de_pad, W_in, b_in.reshape(1, _D))
    table = table.reshape(_NW * 8, _D)

    mesh = plsc.VectorSubcoreMesh(core_axis_name="c", subcore_axis_name="s")
    gather = pl.kernel(
        _gather_body,
        out_type=jax.ShapeDtypeStruct((_E, _D), jnp.float32),
        mesh=mesh,
        scratch_types=[
            pltpu.VMEM((_BLK,), jnp.float32),
            pltpu.VMEM((_NG, _GRP), jnp.int32),
            pltpu.VMEM((2, _BLK, _D), jnp.float32),
            pltpu.SemaphoreType.DMA,
            pltpu.SemaphoreType.DMA,
        ],
    )
    return gather(table, dist_feat)
